# Initial kernel scaffold; baseline (speedup 1.0000x reference)
import jax
import jax.numpy as jnp
from jax.experimental import pallas as pl


def kernel(sequences, table):
    raise NotImplementedError("write your pallas kernel here")


# speedup vs baseline: 1.4160x; 1.4160x over previous
import functools

import jax
import jax.numpy as jnp
import numpy as np
from jax import lax
from jax.experimental import pallas as pl
from jax.experimental.pallas import tpu as pltpu
from jax.experimental.pallas import tpu_sc as plsc

D_MODEL = 32
CHUNK = 128
NBUF = 2
SCALE = np.float32(np.sqrt(np.float32(D_MODEL)))

_NC = 2
_NS = 16
_NW = _NC * _NS


def _make_sc_kernel(n_idx: int):
    assert n_idx % (_NW * CHUNK) == 0
    chunks_per_w = n_idx // (_NW * CHUNK)
    assert chunks_per_w % NBUF == 0
    n_chunk_rows = n_idx // CHUNK

    mesh = plsc.VectorSubcoreMesh(core_axis_name="c", subcore_axis_name="s")

    @functools.partial(
        pl.kernel,
        mesh=mesh,
        out_type=jax.ShapeDtypeStruct((n_idx, D_MODEL), jnp.float32),
        compiler_params=pltpu.CompilerParams(use_tc_tiling_on_sc=False),
        scratch_types=[
            pltpu.VMEM((chunks_per_w, CHUNK), jnp.int32),
            pltpu.VMEM((NBUF, CHUNK, D_MODEL), jnp.float32),
            pltpu.VMEM((NBUF, CHUNK, D_MODEL), jnp.float32),
            pltpu.SemaphoreType.DMA,
            pltpu.SemaphoreType.DMA,
            pltpu.SemaphoreType.DMA,
            pltpu.SemaphoreType.DMA,
        ],
    )
    def k(idx_hbm, table_hbm, out_hbm, idx_v, gbuf, wbuf, gs0, gs1, ws0, ws1):
        gsems = (gs0, gs1)
        wsems = (ws0, ws1)
        wid = lax.axis_index("s") * _NC + lax.axis_index("c")
        chunk_base = wid * chunks_per_w
        row_base = chunk_base * CHUNK

        pltpu.sync_copy(idx_hbm.at[pl.ds(chunk_base, chunks_per_w)], idx_v)

        def fire_gather(g, b):
            pltpu.async_copy(table_hbm.at[idx_v.at[g]], gbuf.at[b], gsems[b])

        def fire_write(g, b):
            pltpu.async_copy(
                wbuf.at[b], out_hbm.at[pl.ds(row_base + g * CHUNK, CHUNK)], wsems[b]
            )

        for b in range(NBUF):
            fire_gather(b, b)

        def body(g0, carry):
            for b in range(NBUF):
                g = g0 + b
                @pl.when(g0 >= NBUF)
                def _():
                    pltpu.make_async_copy(
                        wbuf.at[b],
                        out_hbm.at[pl.ds(row_base, CHUNK)],
                        wsems[b],
                    ).wait()

                pltpu.make_async_copy(
                    table_hbm.at[idx_v.at[g]], gbuf.at[b], gsems[b]
                ).wait()

                for r in range(CHUNK):
                    for c in (0, 16):
                        wbuf[b, r, pl.ds(c, 16)] = gbuf[b, r, pl.ds(c, 16)] * SCALE

                fire_write(g, b)

                @pl.when(g0 + NBUF < chunks_per_w)
                def _():
                    fire_gather(g + NBUF, b)
            return carry

        lax.fori_loop(0, chunks_per_w // NBUF,
                      lambda i, c: body(i * NBUF, c), 0, unroll=False)

        for b in range(NBUF):
            pltpu.make_async_copy(
                wbuf.at[b], out_hbm.at[pl.ds(row_base, CHUNK)], wsems[b]
            ).wait()

    return k


def kernel(sequences, table):
    n_idx = sequences.shape[0] * sequences.shape[1]
    idx2d = sequences.reshape(n_idx // CHUNK, CHUNK).astype(jnp.int32)
    out = _make_sc_kernel(n_idx)(idx2d, table)
    return out.reshape(sequences.shape[0], sequences.shape[1], D_MODEL)
